# baseline (speedup 1.0000x reference)
import jax
import jax.numpy as jnp
from jax.experimental import pallas as pl
from jax.experimental.pallas import tpu as pltpu


def _mem_kernel(q_ref, keys_ref, uq_ref, um_ref, sq_ref, sm_ref, sep_ref,
                comp_ref, s_all, colmax, zsum, uacc, lsum):
    i = pl.program_id(0)
    nb = pl.num_programs(0) // 2
    msl, d = keys_ref.shape
    hw = q_ref.shape[2]
    n = nb * hw

    @pl.when(i == 0)
    def _init():
        colmax[...] = jnp.full_like(colmax[...], -jnp.inf)
        zsum[...] = jnp.zeros_like(zsum[...])
        uacc[...] = jnp.zeros_like(uacc[...])
        lsum[...] = jnp.zeros_like(lsum[...])

    @pl.when(i < nb)
    def _batch_phase():
        b = i
        q = q_ref[0]
        keys = keys_ref[...]

        qnorm = jnp.sqrt(jnp.sum(q * q, axis=0, keepdims=True))
        qn = q / jnp.maximum(qnorm, 1e-12)

        s = jax.lax.dot_general(keys, qn, (((1,), (0,)), ((), ())),
                                preferred_element_type=jnp.float32)
        s_all[:, pl.ds(b * hw, hw)] = s

        smax = jnp.max(s, axis=0, keepdims=True)
        se = jnp.exp(s - smax)
        sm = se / jnp.sum(se, axis=0, keepdims=True)
        sm_ref[...] = jnp.transpose(sm)

        cm = jax.lax.dot_general(keys, sm, (((0,), (0,)), ((), ())),
                                 preferred_element_type=jnp.float32)
        uq_ref[0, :d, :] = qn
        uq_ref[0, d:, :] = cm

        ii = jax.lax.broadcasted_iota(jnp.int32, s.shape, 0)
        idx1 = jnp.min(jnp.where(s == smax, ii, msl), axis=0, keepdims=True)
        oh1 = (ii == idx1).astype(jnp.float32)
        s2 = jnp.where(ii == idx1, -jnp.inf, s)
        m2 = jnp.max(s2, axis=0, keepdims=True)
        idx2 = jnp.min(jnp.where(s2 == m2, ii, msl), axis=0, keepdims=True)
        oh2 = (ii == idx2).astype(jnp.float32)

        k2 = jnp.sum(keys * keys, axis=1, keepdims=True)
        ksum = jnp.sum(keys, axis=1, keepdims=True)
        s_t1 = jnp.sum(oh1 * s, axis=0, keepdims=True)
        s_t2 = jnp.sum(oh2 * s, axis=0, keepdims=True)
        k2_t1 = jnp.sum(oh1 * k2, axis=0, keepdims=True)
        k2_t2 = jnp.sum(oh2 * k2, axis=0, keepdims=True)
        ks_t1 = jnp.sum(oh1 * ksum, axis=0, keepdims=True)
        ks_t2 = jnp.sum(oh2 * ksum, axis=0, keepdims=True)
        qsum = jnp.sum(qn, axis=0, keepdims=True)

        eps = jnp.float32(1e-6)
        dsq_p = 1.0 + k2_t1 - 2.0 * s_t1
        dsq_n = 1.0 + k2_t2 - 2.0 * s_t2
        dp = jnp.sqrt(dsq_p + 2.0 * eps * (qsum - ks_t1) + d * eps * eps)
        dn = jnp.sqrt(dsq_n + 2.0 * eps * (qsum - ks_t2) + d * eps * eps)
        sep_b = jnp.sum(jnp.maximum(dp - dn + 1.0, 0.0))
        comp_b = jnp.sum(dsq_p)
        lsum[...] = lsum[...] + jnp.concatenate(
            [sep_b.reshape(1, 1), comp_b.reshape(1, 1)], axis=1)

        mb = jnp.max(s, axis=1, keepdims=True)
        mnew = jnp.maximum(colmax[...], mb)
        scale = jnp.exp(colmax[...] - mnew)
        e = jnp.exp(s - mnew)
        zsum[...] = zsum[...] * scale + jnp.sum(e, axis=1, keepdims=True)
        sel = e * oh1
        du = jax.lax.dot_general(sel, qn, (((1,), (1,)), ((), ())),
                                 preferred_element_type=jnp.float32)
        uacc[...] = uacc[...] * scale + du
        colmax[...] = mnew

        @pl.when(b == nb - 1)
        def _finalize():
            keys_f = keys_ref[...]
            upd = uacc[...] + keys_f
            un = jnp.sqrt(jnp.sum(upd * upd, axis=1, keepdims=True))
            um_ref[...] = upd / jnp.maximum(un, 1e-12)
            sep_ref[...] = lsum[:, 0:1] / n
            comp_ref[...] = lsum[:, 1:2] / (n * d)

    @pl.when(i >= nb)
    def _sq_phase():
        j = i - nb
        chunk = s_all[:, pl.ds(j * hw, hw)]
        sq = jnp.exp(chunk - colmax[...]) / zsum[...]
        sq_ref[...] = jnp.transpose(sq)


def _build_call(b, c, h, w, msl, interpret=False):
    hw = h * w
    n = b * hw
    f32 = jnp.float32
    last = b - 1
    return pl.pallas_call(
        _mem_kernel,
        grid=(2 * b,),
        in_specs=[
            pl.BlockSpec((1, c, hw), lambda i: (jnp.minimum(i, last), 0, 0)),
            pl.BlockSpec((msl, c), lambda i: (0, 0)),
        ],
        out_specs=[
            pl.BlockSpec((1, 2 * c, hw), lambda i: (jnp.minimum(i, last), 0, 0)),
            pl.BlockSpec((msl, c), lambda i: (0, 0)),
            pl.BlockSpec((hw, msl), lambda i: (jnp.maximum(i - b, 0), 0)),
            pl.BlockSpec((hw, msl), lambda i: (jnp.minimum(i, last), 0)),
            pl.BlockSpec((1, 1), lambda i: (0, 0)),
            pl.BlockSpec((1, 1), lambda i: (0, 0)),
        ],
        out_shape=[
            jax.ShapeDtypeStruct((b, 2 * c, hw), f32),
            jax.ShapeDtypeStruct((msl, c), f32),
            jax.ShapeDtypeStruct((n, msl), f32),
            jax.ShapeDtypeStruct((n, msl), f32),
            jax.ShapeDtypeStruct((1, 1), f32),
            jax.ShapeDtypeStruct((1, 1), f32),
        ],
        scratch_shapes=[
            pltpu.VMEM((msl, n), f32),
            pltpu.VMEM((msl, 1), f32),
            pltpu.VMEM((msl, 1), f32),
            pltpu.VMEM((msl, c), f32),
            pltpu.VMEM((1, 2), f32),
        ],
        compiler_params=pltpu.CompilerParams(
            dimension_semantics=("arbitrary",)),
        interpret=interpret,
    )


@jax.jit
def kernel(query, keys):
    b, c, h, w = query.shape
    msl = keys.shape[0]
    q3 = query.reshape(b, c, h * w)
    uq, um, sq, sm, sep, comp = _build_call(b, c, h, w, msl)(q3, keys)
    updated_query = uq.reshape(b, 2 * c, h, w)
    return (updated_query, um, sq, sm, sep[0, 0], comp[0, 0])


# speedup vs baseline: 1.0633x; 1.0633x over previous
import jax
import jax.numpy as jnp
from jax.experimental import pallas as pl
from jax.experimental.pallas import tpu as pltpu


def _mem_kernel(q_ref, keys_ref, uq_ref, um_ref, sq_ref, sm_ref, sep_ref,
                comp_ref, s_all, colmax, zsum, uacc, lsum):
    i = pl.program_id(0)
    ni = pl.num_programs(0)
    msl, d = keys_ref.shape
    tb = q_ref.shape[2]
    n = ni * tb

    @pl.when(i == 0)
    def _init():
        colmax[...] = jnp.full_like(colmax[...], -jnp.inf)
        zsum[...] = jnp.zeros_like(zsum[...])
        uacc[...] = jnp.zeros_like(uacc[...])
        lsum[...] = jnp.zeros_like(lsum[...])

    q = q_ref[0]
    keys = keys_ref[...]

    qnorm = jnp.sqrt(jnp.sum(q * q, axis=0, keepdims=True))
    qn = q / jnp.maximum(qnorm, 1e-12)

    s = jax.lax.dot_general(keys, qn, (((1,), (0,)), ((), ())),
                            preferred_element_type=jnp.float32)
    s_all[:, pl.ds(i * tb, tb)] = s

    smax = jnp.max(s, axis=0, keepdims=True)
    se = jnp.exp(s - smax)
    sm = se / jnp.sum(se, axis=0, keepdims=True)
    sm_ref[...] = sm

    cm = jax.lax.dot_general(keys, sm, (((0,), (0,)), ((), ())),
                             preferred_element_type=jnp.float32)
    uq_ref[0, :d, :] = qn
    uq_ref[0, d:, :] = cm

    ii = jax.lax.broadcasted_iota(jnp.int32, s.shape, 0)
    idx1 = jnp.min(jnp.where(s == smax, ii, msl), axis=0, keepdims=True)
    oh1 = (ii == idx1).astype(jnp.float32)
    s2 = jnp.where(ii == idx1, -jnp.inf, s)
    m2 = jnp.max(s2, axis=0, keepdims=True)
    idx2 = jnp.min(jnp.where(s2 == m2, ii, msl), axis=0, keepdims=True)
    oh2 = (ii == idx2).astype(jnp.float32)

    k2 = jnp.sum(keys * keys, axis=1, keepdims=True)
    ksum = jnp.sum(keys, axis=1, keepdims=True)
    s_t1 = jnp.sum(oh1 * s, axis=0, keepdims=True)
    s_t2 = jnp.sum(oh2 * s, axis=0, keepdims=True)
    k2_t1 = jnp.sum(oh1 * k2, axis=0, keepdims=True)
    k2_t2 = jnp.sum(oh2 * k2, axis=0, keepdims=True)
    ks_t1 = jnp.sum(oh1 * ksum, axis=0, keepdims=True)
    ks_t2 = jnp.sum(oh2 * ksum, axis=0, keepdims=True)
    qsum = jnp.sum(qn, axis=0, keepdims=True)

    eps = jnp.float32(1e-6)
    dsq_p = 1.0 + k2_t1 - 2.0 * s_t1
    dsq_n = 1.0 + k2_t2 - 2.0 * s_t2
    dp = jnp.sqrt(dsq_p + 2.0 * eps * (qsum - ks_t1) + d * eps * eps)
    dn = jnp.sqrt(dsq_n + 2.0 * eps * (qsum - ks_t2) + d * eps * eps)
    sep_b = jnp.sum(jnp.maximum(dp - dn + 1.0, 0.0))
    comp_b = jnp.sum(dsq_p)
    lsum[...] = lsum[...] + jnp.concatenate(
        [sep_b.reshape(1, 1), comp_b.reshape(1, 1)], axis=1)

    mb = jnp.max(s, axis=1, keepdims=True)
    mnew = jnp.maximum(colmax[...], mb)
    scale = jnp.exp(colmax[...] - mnew)
    e = jnp.exp(s - mnew)
    zsum[...] = zsum[...] * scale + jnp.sum(e, axis=1, keepdims=True)
    sel = e * oh1
    du = jax.lax.dot_general(sel, qn, (((1,), (1,)), ((), ())),
                             preferred_element_type=jnp.float32)
    uacc[...] = uacc[...] * scale + du
    colmax[...] = mnew

    @pl.when(i == ni - 1)
    def _finalize():
        sq_ref[...] = jnp.exp(s_all[...] - colmax[...]) / zsum[...]
        upd = uacc[...] + keys
        un = jnp.sqrt(jnp.sum(upd * upd, axis=1, keepdims=True))
        um_ref[...] = upd / jnp.maximum(un, 1e-12)
        sep_ref[...] = lsum[:, 0:1] / n
        comp_ref[...] = lsum[:, 1:2] / (n * d)


_SPLIT = 2


def _build_call(b, c, h, w, msl, interpret=False):
    hw = h * w
    n = b * hw
    tb = hw // _SPLIT
    f32 = jnp.float32
    return pl.pallas_call(
        _mem_kernel,
        grid=(b * _SPLIT,),
        in_specs=[
            pl.BlockSpec((1, c, tb), lambda i: (i // _SPLIT, 0, i % _SPLIT)),
            pl.BlockSpec((msl, c), lambda i: (0, 0)),
        ],
        out_specs=[
            pl.BlockSpec((1, 2 * c, tb), lambda i: (i // _SPLIT, 0, i % _SPLIT)),
            pl.BlockSpec((msl, c), lambda i: (0, 0)),
            pl.BlockSpec((msl, n), lambda i: (0, 0)),
            pl.BlockSpec((msl, tb), lambda i: (0, i)),
            pl.BlockSpec((1, 1), lambda i: (0, 0)),
            pl.BlockSpec((1, 1), lambda i: (0, 0)),
        ],
        out_shape=[
            jax.ShapeDtypeStruct((b, 2 * c, hw), f32),
            jax.ShapeDtypeStruct((msl, c), f32),
            jax.ShapeDtypeStruct((msl, n), f32),
            jax.ShapeDtypeStruct((msl, n), f32),
            jax.ShapeDtypeStruct((1, 1), f32),
            jax.ShapeDtypeStruct((1, 1), f32),
        ],
        scratch_shapes=[
            pltpu.VMEM((msl, n), f32),
            pltpu.VMEM((msl, 1), f32),
            pltpu.VMEM((msl, 1), f32),
            pltpu.VMEM((msl, c), f32),
            pltpu.VMEM((1, 2), f32),
        ],
        compiler_params=pltpu.CompilerParams(
            dimension_semantics=("arbitrary",)),
        interpret=interpret,
    )


@jax.jit
def kernel(query, keys):
    b, c, h, w = query.shape
    msl = keys.shape[0]
    q3 = query.reshape(b, c, h * w)
    uq, um, sq_t, sm_t, sep, comp = _build_call(b, c, h, w, msl)(q3, keys)
    updated_query = uq.reshape(b, 2 * c, h, w)
    return (updated_query, um, sq_t.T, sm_t.T, sep[0, 0], comp[0, 0])


# speedup vs baseline: 1.1930x; 1.1220x over previous
import jax
import jax.numpy as jnp
from jax.experimental import pallas as pl
from jax.experimental.pallas import tpu as pltpu


def _mem_kernel(q_ref, keys_ref, uq_ref, um_ref, sq_ref, sm_ref, sep_ref,
                comp_ref, s_all, colmax, zsum, uacc, lsum):
    i = pl.program_id(0)
    ni = pl.num_programs(0)
    nsub = q_ref.shape[0]
    msl, d = keys_ref.shape
    tb = q_ref.shape[2]
    n = ni * nsub * tb

    @pl.when(i == 0)
    def _init():
        colmax[...] = jnp.full_like(colmax[...], -jnp.inf)
        zsum[...] = jnp.zeros_like(zsum[...])
        uacc[...] = jnp.zeros_like(uacc[...])
        lsum[...] = jnp.zeros_like(lsum[...])

    keys = keys_ref[...]
    k2 = jnp.sum(keys * keys, axis=1, keepdims=True)
    ksum = jnp.sum(keys, axis=1, keepdims=True)

    for sub in range(nsub):
        q = q_ref[sub]

        qnorm = jnp.sqrt(jnp.sum(q * q, axis=0, keepdims=True))
        qn = q / jnp.maximum(qnorm, 1e-12)

        s = jax.lax.dot_general(keys, qn, (((1,), (0,)), ((), ())),
                                preferred_element_type=jnp.float32)
        s_all[:, pl.ds((i * nsub + sub) * tb, tb)] = s

        smax = jnp.max(s, axis=0, keepdims=True)
        se = jnp.exp(s - smax)
        sm = se / jnp.sum(se, axis=0, keepdims=True)
        sm_ref[:, sub * tb:(sub + 1) * tb] = sm

        cm = jax.lax.dot_general(keys, sm, (((0,), (0,)), ((), ())),
                                 preferred_element_type=jnp.float32)
        uq_ref[sub, :d, :] = qn
        uq_ref[sub, d:, :] = cm

        ii = jax.lax.broadcasted_iota(jnp.int32, s.shape, 0)
        idx1 = jnp.min(jnp.where(s == smax, ii, msl), axis=0, keepdims=True)
        oh1 = (ii == idx1).astype(jnp.float32)
        s2 = jnp.where(ii == idx1, -jnp.inf, s)
        m2 = jnp.max(s2, axis=0, keepdims=True)
        idx2 = jnp.min(jnp.where(s2 == m2, ii, msl), axis=0, keepdims=True)
        oh2 = (ii == idx2).astype(jnp.float32)

        s_t1 = jnp.sum(oh1 * s, axis=0, keepdims=True)
        s_t2 = jnp.sum(oh2 * s, axis=0, keepdims=True)
        k2_t1 = jnp.sum(oh1 * k2, axis=0, keepdims=True)
        k2_t2 = jnp.sum(oh2 * k2, axis=0, keepdims=True)
        ks_t1 = jnp.sum(oh1 * ksum, axis=0, keepdims=True)
        ks_t2 = jnp.sum(oh2 * ksum, axis=0, keepdims=True)
        qsum = jnp.sum(qn, axis=0, keepdims=True)

        eps = jnp.float32(1e-6)
        dsq_p = 1.0 + k2_t1 - 2.0 * s_t1
        dsq_n = 1.0 + k2_t2 - 2.0 * s_t2
        dp = jnp.sqrt(dsq_p + 2.0 * eps * (qsum - ks_t1) + d * eps * eps)
        dn = jnp.sqrt(dsq_n + 2.0 * eps * (qsum - ks_t2) + d * eps * eps)
        sep_b = jnp.sum(jnp.maximum(dp - dn + 1.0, 0.0))
        comp_b = jnp.sum(dsq_p)
        lsum[...] = lsum[...] + jnp.concatenate(
            [sep_b.reshape(1, 1), comp_b.reshape(1, 1)], axis=1)

        mb = jnp.max(s, axis=1, keepdims=True)
        mnew = jnp.maximum(colmax[...], mb)
        scale = jnp.exp(colmax[...] - mnew)
        e = jnp.exp(s - mnew)
        zsum[...] = zsum[...] * scale + jnp.sum(e, axis=1, keepdims=True)
        sel = e * oh1
        du = jax.lax.dot_general(sel, qn, (((1,), (1,)), ((), ())),
                                 preferred_element_type=jnp.float32)
        uacc[...] = uacc[...] * scale + du
        colmax[...] = mnew

    @pl.when(i == ni - 1)
    def _finalize():
        sq_ref[...] = jnp.exp(s_all[...] - colmax[...]) / zsum[...]
        upd = uacc[...] + keys
        un = jnp.sqrt(jnp.sum(upd * upd, axis=1, keepdims=True))
        um_ref[...] = upd / jnp.maximum(un, 1e-12)
        sep_ref[...] = lsum[:, 0:1] / n
        comp_ref[...] = lsum[:, 1:2] / (n * d)


_MERGE = 2


def _build_call(b, c, h, w, msl, interpret=False):
    hw = h * w
    n = b * hw
    f32 = jnp.float32
    return pl.pallas_call(
        _mem_kernel,
        grid=(b // _MERGE,),
        in_specs=[
            pl.BlockSpec((_MERGE, c, hw), lambda i: (i, 0, 0)),
            pl.BlockSpec((msl, c), lambda i: (0, 0)),
        ],
        out_specs=[
            pl.BlockSpec((_MERGE, 2 * c, hw), lambda i: (i, 0, 0)),
            pl.BlockSpec((msl, c), lambda i: (0, 0)),
            pl.BlockSpec((msl, n), lambda i: (0, 0)),
            pl.BlockSpec((msl, _MERGE * hw), lambda i: (0, i)),
            pl.BlockSpec((1, 1), lambda i: (0, 0)),
            pl.BlockSpec((1, 1), lambda i: (0, 0)),
        ],
        out_shape=[
            jax.ShapeDtypeStruct((b, 2 * c, hw), f32),
            jax.ShapeDtypeStruct((msl, c), f32),
            jax.ShapeDtypeStruct((msl, n), f32),
            jax.ShapeDtypeStruct((msl, n), f32),
            jax.ShapeDtypeStruct((1, 1), f32),
            jax.ShapeDtypeStruct((1, 1), f32),
        ],
        scratch_shapes=[
            pltpu.VMEM((msl, n), f32),
            pltpu.VMEM((msl, 1), f32),
            pltpu.VMEM((msl, 1), f32),
            pltpu.VMEM((msl, c), f32),
            pltpu.VMEM((1, 2), f32),
        ],
        compiler_params=pltpu.CompilerParams(
            dimension_semantics=("arbitrary",)),
        interpret=interpret,
    )


@jax.jit
def kernel(query, keys):
    b, c, h, w = query.shape
    msl = keys.shape[0]
    q3 = query.reshape(b, c, h * w)
    uq, um, sq_t, sm_t, sep, comp = _build_call(b, c, h, w, msl)(q3, keys)
    updated_query = uq.reshape(b, 2 * c, h, w)
    return (updated_query, um, sq_t.T, sm_t.T, sep[0, 0], comp[0, 0])


# speedup vs baseline: 1.2015x; 1.0071x over previous
import jax
import jax.numpy as jnp
from jax.experimental import pallas as pl
from jax.experimental.pallas import tpu as pltpu


def _mem_kernel(q_ref, keys_ref, uq_ref, um_ref, sq_ref, sm_ref, sep_ref,
                comp_ref, s_all, colmax, zsum, uacc, lsum):
    i = pl.program_id(0)
    ni = pl.num_programs(0)
    nsub = q_ref.shape[0]
    msl, d = keys_ref.shape
    tb = q_ref.shape[2]
    n = ni * nsub * tb

    @pl.when(i == 0)
    def _init():
        colmax[...] = jnp.full_like(colmax[...], -jnp.inf)
        zsum[...] = jnp.zeros_like(zsum[...])
        uacc[...] = jnp.zeros_like(uacc[...])
        lsum[...] = jnp.zeros_like(lsum[...])

    keys = keys_ref[...]
    k2 = jnp.sum(keys * keys, axis=1, keepdims=True)
    ksum = jnp.sum(keys, axis=1, keepdims=True)

    for sub in range(nsub):
        q = q_ref[sub]

        qnorm = jnp.sqrt(jnp.sum(q * q, axis=0, keepdims=True))
        qn = q / jnp.maximum(qnorm, 1e-12)

        s = jax.lax.dot_general(keys, qn, (((1,), (0,)), ((), ())),
                                preferred_element_type=jnp.float32)
        s_all[:, pl.ds((i * nsub + sub) * tb, tb)] = s

        smax = jnp.max(s, axis=0, keepdims=True)
        se = jnp.exp(s - smax)
        sm = se / jnp.sum(se, axis=0, keepdims=True)
        sm_ref[:, sub * tb:(sub + 1) * tb] = sm

        cm = jax.lax.dot_general(keys, sm, (((0,), (0,)), ((), ())),
                                 preferred_element_type=jnp.float32)
        uq_ref[sub, :d, :] = qn
        uq_ref[sub, d:, :] = cm

        ii = jax.lax.broadcasted_iota(jnp.int32, s.shape, 0)
        idx1 = jnp.min(jnp.where(s == smax, ii, msl), axis=0, keepdims=True)
        oh1 = (ii == idx1).astype(jnp.float32)
        s2 = jnp.where(ii == idx1, -jnp.inf, s)
        m2 = jnp.max(s2, axis=0, keepdims=True)
        idx2 = jnp.min(jnp.where(s2 == m2, ii, msl), axis=0, keepdims=True)
        oh2 = (ii == idx2).astype(jnp.float32)

        s_t1 = jnp.sum(oh1 * s, axis=0, keepdims=True)
        s_t2 = jnp.sum(oh2 * s, axis=0, keepdims=True)
        k2_t1 = jnp.sum(oh1 * k2, axis=0, keepdims=True)
        k2_t2 = jnp.sum(oh2 * k2, axis=0, keepdims=True)
        ks_t1 = jnp.sum(oh1 * ksum, axis=0, keepdims=True)
        ks_t2 = jnp.sum(oh2 * ksum, axis=0, keepdims=True)
        qsum = jnp.sum(qn, axis=0, keepdims=True)

        eps = jnp.float32(1e-6)
        dsq_p = 1.0 + k2_t1 - 2.0 * s_t1
        dsq_n = 1.0 + k2_t2 - 2.0 * s_t2
        dp = jnp.sqrt(dsq_p + 2.0 * eps * (qsum - ks_t1) + d * eps * eps)
        dn = jnp.sqrt(dsq_n + 2.0 * eps * (qsum - ks_t2) + d * eps * eps)
        sep_b = jnp.sum(jnp.maximum(dp - dn + 1.0, 0.0))
        comp_b = jnp.sum(dsq_p)
        lsum[...] = lsum[...] + jnp.concatenate(
            [sep_b.reshape(1, 1), comp_b.reshape(1, 1)], axis=1)

        mb = jnp.max(s, axis=1, keepdims=True)
        mnew = jnp.maximum(colmax[...], mb)
        scale = jnp.exp(colmax[...] - mnew)
        e = jnp.exp(s - mnew)
        zsum[...] = zsum[...] * scale + jnp.sum(e, axis=1, keepdims=True)
        sel = e * oh1
        du = jax.lax.dot_general(sel, qn, (((1,), (1,)), ((), ())),
                                 preferred_element_type=jnp.float32)
        uacc[...] = uacc[...] * scale + du
        colmax[...] = mnew

    @pl.when(i == ni - 1)
    def _finalize():
        sq_ref[...] = jnp.exp(s_all[...] - colmax[...]) / zsum[...]
        upd = uacc[...] + keys
        un = jnp.sqrt(jnp.sum(upd * upd, axis=1, keepdims=True))
        um_ref[...] = upd / jnp.maximum(un, 1e-12)
        sep_ref[...] = lsum[:, 0:1] / n
        comp_ref[...] = lsum[:, 1:2] / (n * d)


_MERGE = 4


def _build_call(b, c, h, w, msl, interpret=False):
    hw = h * w
    n = b * hw
    f32 = jnp.float32
    return pl.pallas_call(
        _mem_kernel,
        grid=(b // _MERGE,),
        in_specs=[
            pl.BlockSpec((_MERGE, c, hw), lambda i: (i, 0, 0)),
            pl.BlockSpec((msl, c), lambda i: (0, 0)),
        ],
        out_specs=[
            pl.BlockSpec((_MERGE, 2 * c, hw), lambda i: (i, 0, 0)),
            pl.BlockSpec((msl, c), lambda i: (0, 0)),
            pl.BlockSpec((msl, n), lambda i: (0, 0)),
            pl.BlockSpec((msl, _MERGE * hw), lambda i: (0, i)),
            pl.BlockSpec((1, 1), lambda i: (0, 0)),
            pl.BlockSpec((1, 1), lambda i: (0, 0)),
        ],
        out_shape=[
            jax.ShapeDtypeStruct((b, 2 * c, hw), f32),
            jax.ShapeDtypeStruct((msl, c), f32),
            jax.ShapeDtypeStruct((msl, n), f32),
            jax.ShapeDtypeStruct((msl, n), f32),
            jax.ShapeDtypeStruct((1, 1), f32),
            jax.ShapeDtypeStruct((1, 1), f32),
        ],
        scratch_shapes=[
            pltpu.VMEM((msl, n), f32),
            pltpu.VMEM((msl, 1), f32),
            pltpu.VMEM((msl, 1), f32),
            pltpu.VMEM((msl, c), f32),
            pltpu.VMEM((1, 2), f32),
        ],
        compiler_params=pltpu.CompilerParams(
            dimension_semantics=("arbitrary",)),
        interpret=interpret,
    )


@jax.jit
def kernel(query, keys):
    b, c, h, w = query.shape
    msl = keys.shape[0]
    q3 = query.reshape(b, c, h * w)
    uq, um, sq_t, sm_t, sep, comp = _build_call(b, c, h, w, msl)(q3, keys)
    updated_query = uq.reshape(b, 2 * c, h, w)
    return (updated_query, um, sq_t.T, sm_t.T, sep[0, 0], comp[0, 0])
